# baseline (speedup 1.0000x reference)
import functools

import jax
import jax.numpy as jnp
from jax import lax
from jax.experimental import pallas as pl
from jax.experimental.pallas import tpu as pltpu
from jax.experimental.pallas import tpu_sc as plsc

NN = 10000
EE = 320000
CC = 128
TT = 3
HC = CC // 2

CHUNK = 48
SB = 336
NCH = SB // CHUNK
TILES = 16
EDGES_PER_TILE = 20160
NSB = EDGES_PER_TILE // SB
E_PAD = EDGES_PER_TILE * TILES
E_ALLOC = E_PAD + SB
STRIPE = 1888
SLOTS = STRIPE * TILES
WZ = 16
TRASH = TT * NN

BR = 400
NB = NN // BR
_PREC = lax.Precision.HIGHEST


def _sc_edge_pass(table, src_p, dst_p, typ_p):
    mesh = plsc.VectorSubcoreMesh(core_axis_name="c", subcore_axis_name="s")

    @functools.partial(
        pl.kernel,
        out_type=jax.ShapeDtypeStruct((2, SLOTS, HC), jnp.float32),
        mesh=mesh,
        scratch_types=[
            pltpu.VMEM((2, SB), jnp.int32),
            pltpu.VMEM((2, SB), jnp.int32),
            pltpu.VMEM((2, SB), jnp.int32),
            pltpu.VMEM((CHUNK,), jnp.int32),
            pltpu.VMEM((CHUNK,), jnp.int32),
            pltpu.VMEM((CHUNK,), jnp.int32),
            pltpu.VMEM((CHUNK,), jnp.int32),
            pltpu.VMEM((2, CHUNK, HC), jnp.float32),
            pltpu.VMEM((WZ, HC), jnp.float32),
            pltpu.VMEM_SHARED((SLOTS, HC), jnp.float32),
            pltpu.SemaphoreType.DMA,
            pltpu.SemaphoreType.DMA,
            pltpu.SemaphoreType.DMA,
            pltpu.SemaphoreType.DMA,
            pltpu.SemaphoreType.DMA,
            pltpu.SemaphoreType.DMA,
        ],
        compiler_params=pltpu.CompilerParams(use_tc_tiling_on_sc=False),
    )
    def k(table_h, src_h, dst_h, typ_h, out_h,
          src_v, dst_v, typ_v, gidx0_v, gidx1_v, didx0_v, didx1_v,
          rows_v, wrows_v,
          acc, sem_i0, sem_i1, sem_g0, sem_g1, sem_s0, sem_s1):
        c = lax.axis_index("c")
        s = lax.axis_index("s")
        gidx = (gidx0_v, gidx1_v)
        didx = (didx0_v, didx1_v)
        sem_i = (sem_i0, sem_i1)
        sem_g = (sem_g0, sem_g1)
        sem_s = (sem_s0, sem_s1)
        stripe_base = s * STRIPE

        def zrow(i, carry):
            for j in range(HC // 16):
                wrows_v[i, pl.ds(j * 16, 16)] = jnp.zeros((16,), jnp.float32)
            return carry
        lax.fori_loop(0, WZ, zrow, 0)

        def zacc(kk, carry):
            pltpu.sync_copy(
                wrows_v, acc.at[pl.ds(stripe_base + kk * WZ, WZ)])
            return carry
        lax.fori_loop(0, STRIPE // WZ, zacc, 0)
        plsc.subcore_barrier()

        def issue_idx_load(sb, buf):
            base = s * EDGES_PER_TILE + sb * SB
            pltpu.async_copy(src_h.at[pl.ds(base, SB)], src_v.at[buf],
                             sem_i[buf])
            pltpu.async_copy(dst_h.at[pl.ds(base, SB)], dst_v.at[buf],
                             sem_i[buf])
            pltpu.async_copy(typ_h.at[pl.ds(base, SB)], typ_v.at[buf],
                             sem_i[buf])

        def wait_idx_load(buf):
            for ref in (src_v, dst_v, typ_v):
                pltpu.make_async_copy(src_h.at[pl.ds(0, SB)], ref.at[buf],
                                      sem_i[buf]).wait()

        def process_superblock(buf, rb_base, state):
            gdesc, sdesc = state
            for ch in range(NCH):
                rb = (rb_base + ch) & 1
                if sdesc[rb] is not None:
                    sdesc[rb].wait()
                    sdesc[rb] = None
                for g in range(CHUNK // 16):
                    off = pl.ds(ch * CHUNK + g * 16, 16)
                    sl = pl.ds(g * 16, 16)
                    t16 = typ_v[buf, off]
                    gidx[rb][sl] = (t16 * NN + src_v[buf, off]) * 2 + c
                    didx[rb][sl] = t16 * NN + dst_v[buf, off]
                if gdesc[rb ^ 1] is not None:
                    gdesc[rb ^ 1].wait()
                    gdesc[rb ^ 1] = None
                    sdesc[rb ^ 1] = pltpu.async_copy(
                        rows_v.at[rb ^ 1], acc.at[didx[rb ^ 1]],
                        sem_s[rb ^ 1], add=True)
                gdesc[rb] = pltpu.async_copy(
                    table_h.at[gidx[rb]], rows_v.at[rb], sem_g[rb])
            return gdesc, sdesc

        def drain(state):
            gdesc, sdesc = state
            for rb in (0, 1):
                if gdesc[rb] is not None:
                    if sdesc[rb] is not None:
                        sdesc[rb].wait()
                    gdesc[rb].wait()
                    sdesc[rb] = pltpu.async_copy(
                        rows_v.at[rb], acc.at[didx[rb]],
                        sem_s[rb], add=True)
                    gdesc[rb] = None
            for rb in (0, 1):
                if sdesc[rb] is not None:
                    sdesc[rb].wait()

        issue_idx_load(0, 0)

        def pair_body(i, carry):
            sb0 = 2 * i
            wait_idx_load(0)
            issue_idx_load(sb0 + 1, 1)
            state = process_superblock(0, 0, ([None, None], [None, None]))
            wait_idx_load(1)
            issue_idx_load(sb0 + 2, 0)
            state = process_superblock(1, NCH & 1, state)
            drain(state)
            return carry
        lax.fori_loop(0, NSB // 2, pair_body, 0)
        wait_idx_load(0)
        plsc.subcore_barrier()

        pltpu.sync_copy(acc.at[pl.ds(stripe_base, STRIPE)],
                        out_h.at[c, pl.ds(stripe_base, STRIPE)])

    return k(table, src_p, dst_p, typ_p)


def _msg_matmul(h, W0):
    def body(h_ref, w_ref, o_ref):
        o_ref[0] = lax.dot_general(
            h_ref[...], w_ref[0], (((1,), (0,)), ((), ())),
            preferred_element_type=jnp.float32, precision=_PREC)
    return pl.pallas_call(
        body,
        grid=(TT, NB),
        in_specs=[
            pl.BlockSpec((BR, CC), lambda t, r: (r, 0)),
            pl.BlockSpec((1, CC, CC), lambda t, r: (t, 0, 0)),
        ],
        out_specs=pl.BlockSpec((1, BR, CC), lambda t, r: (t, r, 0)),
        out_shape=jax.ShapeDtypeStruct((TT, NN, CC), jnp.float32),
    )(h, W0)


def _gh_step1(h, Whh, bhh):
    def body(h_ref, whh_ref, bhh_ref, o_ref):
        o_ref[0] = lax.dot_general(
            h_ref[...], whh_ref[0], (((1,), (1,)), ((), ())),
            preferred_element_type=jnp.float32,
            precision=_PREC) + bhh_ref[0, 0]
    return pl.pallas_call(
        body,
        grid=(TT, NB),
        in_specs=[
            pl.BlockSpec((BR, CC), lambda t, r: (r, 0)),
            pl.BlockSpec((1, 3 * CC, CC), lambda t, r: (t, 0, 0)),
            pl.BlockSpec((1, 1, 3 * CC), lambda t, r: (t, 0, 0)),
        ],
        out_specs=pl.BlockSpec((1, BR, 3 * CC), lambda t, r: (t, r, 0)),
        out_shape=jax.ShapeDtypeStruct((TT, NN, 3 * CC), jnp.float32),
    )(h, Whh, bhh.reshape(TT, 1, 3 * CC))


def _gh_step2(x1, Whh, bhh):
    def body(x_ref, whh_ref, bhh_ref, o_ref):
        o_ref[0] = lax.dot_general(
            x_ref[0], whh_ref[0], (((1,), (1,)), ((), ())),
            preferred_element_type=jnp.float32,
            precision=_PREC) + bhh_ref[0, 0]
    return pl.pallas_call(
        body,
        grid=(TT, NB),
        in_specs=[
            pl.BlockSpec((1, BR, CC), lambda t, r: (t, r, 0)),
            pl.BlockSpec((1, 3 * CC, CC), lambda t, r: (t, 0, 0)),
            pl.BlockSpec((1, 1, 3 * CC), lambda t, r: (t, 0, 0)),
        ],
        out_specs=pl.BlockSpec((1, BR, 3 * CC), lambda t, r: (t, r, 0)),
        out_shape=jax.ShapeDtypeStruct((TT, NN, 3 * CC), jnp.float32),
    )(x1, Whh, bhh.reshape(TT, 1, 3 * CC))


def _gru_combine(alo, ahi, gh, hp, wih, bih):
    gi = lax.dot_general(alo, wih[:, :HC], (((1,), (1,)), ((), ())),
                         preferred_element_type=jnp.float32,
                         precision=_PREC)
    gi = gi + lax.dot_general(ahi, wih[:, HC:], (((1,), (1,)), ((), ())),
                              preferred_element_type=jnp.float32,
                              precision=_PREC) + bih
    r = jax.nn.sigmoid(gi[:, :CC] + gh[:, :CC])
    z = jax.nn.sigmoid(gi[:, CC:2 * CC] + gh[:, CC:2 * CC])
    n = jnp.tanh(gi[:, 2 * CC:] + r * gh[:, 2 * CC:])
    return (1.0 - z) * n + z * hp


def _gru_and_msg(agg, h, gh1, Wih, bih, W1):
    def body(alo_ref, ahi_ref, h_ref, gh_ref, wih_ref, bih_ref, w1_ref,
             x1_ref, m1_ref):
        x1 = _gru_combine(alo_ref[0], ahi_ref[0], gh_ref[0], h_ref[...],
                          wih_ref[0], bih_ref[0, 0])
        x1_ref[0] = x1
        m1_ref[0] = lax.dot_general(
            x1, w1_ref[0], (((1,), (0,)), ((), ())),
            preferred_element_type=jnp.float32, precision=_PREC)
    return pl.pallas_call(
        body,
        grid=(TT, NB),
        in_specs=[
            pl.BlockSpec((1, BR, HC), lambda t, r: (0, t * NB + r, 0)),
            pl.BlockSpec((1, BR, HC), lambda t, r: (1, t * NB + r, 0)),
            pl.BlockSpec((BR, CC), lambda t, r: (r, 0)),
            pl.BlockSpec((1, BR, 3 * CC), lambda t, r: (t, r, 0)),
            pl.BlockSpec((1, 3 * CC, CC), lambda t, r: (t, 0, 0)),
            pl.BlockSpec((1, 1, 3 * CC), lambda t, r: (t, 0, 0)),
            pl.BlockSpec((1, CC, CC), lambda t, r: (t, 0, 0)),
        ],
        out_specs=[
            pl.BlockSpec((1, BR, CC), lambda t, r: (t, r, 0)),
            pl.BlockSpec((1, BR, CC), lambda t, r: (t, r, 0)),
        ],
        out_shape=[
            jax.ShapeDtypeStruct((TT, NN, CC), jnp.float32),
            jax.ShapeDtypeStruct((TT, NN, CC), jnp.float32),
        ],
    )(agg, agg, h, gh1, Wih, bih.reshape(TT, 1, 3 * CC), W1)


def _final(agg1, x1, gh2, h, Wih, bih, ln_g, ln_b):
    def body(a0l_ref, a0h_ref, a1l_ref, a1h_ref, a2l_ref, a2h_ref,
             x1_ref, gh_ref, h_ref, wih_ref,
             bih_ref, lng_ref, lnb_ref, y_ref):
        a_refs = ((a0l_ref, a0h_ref), (a1l_ref, a1h_ref), (a2l_ref, a2h_ref))
        acc = h_ref[...]
        for t in range(TT):
            acc = acc + _gru_combine(a_refs[t][0][0], a_refs[t][1][0],
                                     gh_ref[t], x1_ref[t],
                                     wih_ref[t], bih_ref[t])
        mu = jnp.mean(acc, axis=1, keepdims=True)
        d = acc - mu
        var = jnp.mean(d * d, axis=1, keepdims=True)
        y = d * lax.rsqrt(var + 1e-5) * lng_ref[...] + lnb_ref[...]
        y_ref[...] = jnp.where(y >= 0, y, 0.1 * y)

    def agg_spec(t, half):
        return pl.BlockSpec((1, BR, HC),
                            lambda r, t=t, half=half: (half, t * NB + r, 0))
    return pl.pallas_call(
        body,
        grid=(NB,),
        in_specs=[
            agg_spec(0, 0), agg_spec(0, 1),
            agg_spec(1, 0), agg_spec(1, 1),
            agg_spec(2, 0), agg_spec(2, 1),
            pl.BlockSpec((TT, BR, CC), lambda r: (0, r, 0)),
            pl.BlockSpec((TT, BR, 3 * CC), lambda r: (0, r, 0)),
            pl.BlockSpec((BR, CC), lambda r: (r, 0)),
            pl.BlockSpec((TT, 3 * CC, CC), lambda r: (0, 0, 0)),
            pl.BlockSpec((TT, 3 * CC), lambda r: (0, 0)),
            pl.BlockSpec((1, CC), lambda r: (0, 0)),
            pl.BlockSpec((1, CC), lambda r: (0, 0)),
        ],
        out_specs=pl.BlockSpec((BR, CC), lambda r: (r, 0)),
        out_shape=jax.ShapeDtypeStruct((NN, CC), jnp.float32),
    )(agg1, agg1, agg1, agg1, agg1, agg1, x1, gh2, h, Wih, bih, ln_g, ln_b)


def kernel(h, edge_index, edge_type, W, Wih, Whh, bih, bhh, ln_g, ln_b):
    src = edge_index[0].astype(jnp.int32)
    dst = edge_index[1].astype(jnp.int32)
    typ = edge_type.astype(jnp.int32)
    npad = E_ALLOC - EE
    src_p = jnp.concatenate([src, jnp.zeros((npad,), jnp.int32)])
    dst_p = jnp.concatenate([dst, jnp.full((npad,), TRASH, jnp.int32)])
    typ_p = jnp.concatenate([typ, jnp.zeros((npad,), jnp.int32)])

    m0 = _msg_matmul(h, W[:, 0])
    agg0 = _sc_edge_pass(m0.reshape(2 * TT * NN, HC),
                         src_p, dst_p, typ_p)
    gh1 = _gh_step1(h, Whh, bhh)
    x1, m1 = _gru_and_msg(agg0, h, gh1, Wih, bih, W[:, 1])
    agg1 = _sc_edge_pass(m1.reshape(2 * TT * NN, HC),
                         src_p, dst_p, typ_p)
    gh2 = _gh_step2(x1, Whh, bhh)
    return _final(agg1, x1, gh2, h, Wih, bih,
                  ln_g.reshape(1, CC), ln_b.reshape(1, CC))


# speedup vs baseline: 1.0715x; 1.0715x over previous
import functools

import jax
import jax.numpy as jnp
from jax import lax
from jax.experimental import pallas as pl
from jax.experimental.pallas import tpu as pltpu
from jax.experimental.pallas import tpu_sc as plsc

NN = 10000
EE = 320000
CC = 128
TT = 3
HC = CC // 2

CHUNK = 48
SB = 336
NCH = SB // CHUNK
TILES = 16
EDGES_PER_TILE = 20160
NSB = EDGES_PER_TILE // SB
E_PAD = EDGES_PER_TILE * TILES
E_ALLOC = E_PAD + SB
STRIPE = 1876
SLOTS = STRIPE * TILES
WZ = 28
TRASH = TT * NN

BR = 400
NB = NN // BR
_PREC = lax.Precision.HIGHEST


def _sc_edge_pass(table, src_p, dst_p, typ_p):
    mesh = plsc.VectorSubcoreMesh(core_axis_name="c", subcore_axis_name="s")

    @functools.partial(
        pl.kernel,
        out_type=jax.ShapeDtypeStruct((2, SLOTS, HC), jnp.float32),
        mesh=mesh,
        scratch_types=[
            pltpu.VMEM((2, SB), jnp.int32),
            pltpu.VMEM((2, SB), jnp.int32),
            pltpu.VMEM((2, SB), jnp.int32),
            pltpu.VMEM((CHUNK,), jnp.int32),
            pltpu.VMEM((CHUNK,), jnp.int32),
            pltpu.VMEM((CHUNK,), jnp.int32),
            pltpu.VMEM((CHUNK,), jnp.int32),
            pltpu.VMEM((2, CHUNK, HC), jnp.float32),
            pltpu.VMEM((WZ, HC), jnp.float32),
            pltpu.VMEM_SHARED((SLOTS, HC), jnp.float32),
            pltpu.SemaphoreType.DMA,
            pltpu.SemaphoreType.DMA,
            pltpu.SemaphoreType.DMA,
            pltpu.SemaphoreType.DMA,
            pltpu.SemaphoreType.DMA,
            pltpu.SemaphoreType.DMA,
        ],
        compiler_params=pltpu.CompilerParams(use_tc_tiling_on_sc=False),
    )
    def k(table_h, src_h, dst_h, typ_h, out_h,
          src_v, dst_v, typ_v, gidx0_v, gidx1_v, didx0_v, didx1_v,
          rows_v, wrows_v,
          acc, sem_i0, sem_i1, sem_g0, sem_g1, sem_s0, sem_s1):
        c = lax.axis_index("c")
        s = lax.axis_index("s")
        gidx = (gidx0_v, gidx1_v)
        didx = (didx0_v, didx1_v)
        sem_i = (sem_i0, sem_i1)
        sem_g = (sem_g0, sem_g1)
        sem_s = (sem_s0, sem_s1)
        stripe_base = s * STRIPE

        def zrow(i, carry):
            for j in range(HC // 16):
                wrows_v[i, pl.ds(j * 16, 16)] = jnp.zeros((16,), jnp.float32)
            return carry
        lax.fori_loop(0, WZ, zrow, 0)

        def zacc(kk, carry):
            pltpu.sync_copy(
                wrows_v, acc.at[pl.ds(stripe_base + kk * WZ, WZ)])
            return carry
        lax.fori_loop(0, STRIPE // WZ, zacc, 0)
        plsc.subcore_barrier()

        def issue_idx_load(sb, buf):
            base = s * EDGES_PER_TILE + sb * SB
            pltpu.async_copy(src_h.at[pl.ds(base, SB)], src_v.at[buf],
                             sem_i[buf])
            pltpu.async_copy(dst_h.at[pl.ds(base, SB)], dst_v.at[buf],
                             sem_i[buf])
            pltpu.async_copy(typ_h.at[pl.ds(base, SB)], typ_v.at[buf],
                             sem_i[buf])

        def wait_idx_load(buf):
            for ref in (src_v, dst_v, typ_v):
                pltpu.make_async_copy(src_h.at[pl.ds(0, SB)], ref.at[buf],
                                      sem_i[buf]).wait()

        def process_superblock(buf, rb_base, state):
            gdesc, sdesc = state
            for ch in range(NCH):
                rb = (rb_base + ch) & 1
                if sdesc[rb] is not None:
                    sdesc[rb].wait()
                    sdesc[rb] = None
                for g in range(CHUNK // 16):
                    off = pl.ds(ch * CHUNK + g * 16, 16)
                    sl = pl.ds(g * 16, 16)
                    t16 = typ_v[buf, off]
                    gidx[rb][sl] = (t16 * NN + src_v[buf, off]) * 2 + c
                    didx[rb][sl] = t16 * NN + dst_v[buf, off]
                if gdesc[rb ^ 1] is not None:
                    gdesc[rb ^ 1].wait()
                    gdesc[rb ^ 1] = None
                    sdesc[rb ^ 1] = pltpu.async_copy(
                        rows_v.at[rb ^ 1], acc.at[didx[rb ^ 1]],
                        sem_s[rb ^ 1], add=True)
                gdesc[rb] = pltpu.async_copy(
                    table_h.at[gidx[rb]], rows_v.at[rb], sem_g[rb])
            return gdesc, sdesc

        def drain(state):
            gdesc, sdesc = state
            for rb in (0, 1):
                if gdesc[rb] is not None:
                    if sdesc[rb] is not None:
                        sdesc[rb].wait()
                    gdesc[rb].wait()
                    sdesc[rb] = pltpu.async_copy(
                        rows_v.at[rb], acc.at[didx[rb]],
                        sem_s[rb], add=True)
                    gdesc[rb] = None
            for rb in (0, 1):
                if sdesc[rb] is not None:
                    sdesc[rb].wait()

        issue_idx_load(0, 0)

        def pair_body(i, carry):
            sb0 = 2 * i
            wait_idx_load(0)
            issue_idx_load(sb0 + 1, 1)
            state = process_superblock(0, 0, ([None, None], [None, None]))
            wait_idx_load(1)
            issue_idx_load(sb0 + 2, 0)
            state = process_superblock(1, NCH & 1, state)
            drain(state)
            return carry
        lax.fori_loop(0, NSB // 2, pair_body, 0)
        wait_idx_load(0)
        plsc.subcore_barrier()

        pltpu.sync_copy(acc.at[pl.ds(stripe_base, STRIPE)],
                        out_h.at[c, pl.ds(stripe_base, STRIPE)])

    return k(table, src_p, dst_p, typ_p)


def _msg_matmul(h, W0):
    def body(h_ref, w_ref, o_ref):
        o_ref[0] = lax.dot_general(
            h_ref[...], w_ref[0], (((1,), (0,)), ((), ())),
            preferred_element_type=jnp.float32, precision=_PREC)
    return pl.pallas_call(
        body,
        grid=(TT, NB),
        in_specs=[
            pl.BlockSpec((BR, CC), lambda t, r: (r, 0)),
            pl.BlockSpec((1, CC, CC), lambda t, r: (t, 0, 0)),
        ],
        out_specs=pl.BlockSpec((1, BR, CC), lambda t, r: (t, r, 0)),
        out_shape=jax.ShapeDtypeStruct((TT, NN, CC), jnp.float32),
    )(h, W0)


def _gh_step1(h, Whh, bhh):
    def body(h_ref, whh_ref, bhh_ref, o_ref):
        o_ref[0] = lax.dot_general(
            h_ref[...], whh_ref[0], (((1,), (1,)), ((), ())),
            preferred_element_type=jnp.float32,
            precision=_PREC) + bhh_ref[0, 0]
    return pl.pallas_call(
        body,
        grid=(TT, NB),
        in_specs=[
            pl.BlockSpec((BR, CC), lambda t, r: (r, 0)),
            pl.BlockSpec((1, 3 * CC, CC), lambda t, r: (t, 0, 0)),
            pl.BlockSpec((1, 1, 3 * CC), lambda t, r: (t, 0, 0)),
        ],
        out_specs=pl.BlockSpec((1, BR, 3 * CC), lambda t, r: (t, r, 0)),
        out_shape=jax.ShapeDtypeStruct((TT, NN, 3 * CC), jnp.float32),
    )(h, Whh, bhh.reshape(TT, 1, 3 * CC))


def _gh_step2(x1, Whh, bhh):
    def body(x_ref, whh_ref, bhh_ref, o_ref):
        o_ref[0] = lax.dot_general(
            x_ref[0], whh_ref[0], (((1,), (1,)), ((), ())),
            preferred_element_type=jnp.float32,
            precision=_PREC) + bhh_ref[0, 0]
    return pl.pallas_call(
        body,
        grid=(TT, NB),
        in_specs=[
            pl.BlockSpec((1, BR, CC), lambda t, r: (t, r, 0)),
            pl.BlockSpec((1, 3 * CC, CC), lambda t, r: (t, 0, 0)),
            pl.BlockSpec((1, 1, 3 * CC), lambda t, r: (t, 0, 0)),
        ],
        out_specs=pl.BlockSpec((1, BR, 3 * CC), lambda t, r: (t, r, 0)),
        out_shape=jax.ShapeDtypeStruct((TT, NN, 3 * CC), jnp.float32),
    )(x1, Whh, bhh.reshape(TT, 1, 3 * CC))


def _gru_combine(alo, ahi, gh, hp, wih, bih):
    a = jnp.concatenate((alo, ahi), axis=1)
    gi = lax.dot_general(a, wih, (((1,), (1,)), ((), ())),
                         preferred_element_type=jnp.float32,
                         precision=_PREC) + bih
    r = jax.nn.sigmoid(gi[:, :CC] + gh[:, :CC])
    z = jax.nn.sigmoid(gi[:, CC:2 * CC] + gh[:, CC:2 * CC])
    n = jnp.tanh(gi[:, 2 * CC:] + r * gh[:, 2 * CC:])
    return (1.0 - z) * n + z * hp


def _gru_and_msg(agg, h, gh1, Wih, bih, W1):
    def body(alo_ref, ahi_ref, h_ref, gh_ref, wih_ref, bih_ref, w1_ref,
             x1_ref, m1_ref):
        x1 = _gru_combine(alo_ref[0], ahi_ref[0], gh_ref[0], h_ref[...],
                          wih_ref[0], bih_ref[0, 0])
        x1_ref[0] = x1
        m1_ref[0] = lax.dot_general(
            x1, w1_ref[0], (((1,), (0,)), ((), ())),
            preferred_element_type=jnp.float32, precision=_PREC)
    return pl.pallas_call(
        body,
        grid=(TT, NB),
        in_specs=[
            pl.BlockSpec((1, BR, HC), lambda t, r: (0, t * NB + r, 0)),
            pl.BlockSpec((1, BR, HC), lambda t, r: (1, t * NB + r, 0)),
            pl.BlockSpec((BR, CC), lambda t, r: (r, 0)),
            pl.BlockSpec((1, BR, 3 * CC), lambda t, r: (t, r, 0)),
            pl.BlockSpec((1, 3 * CC, CC), lambda t, r: (t, 0, 0)),
            pl.BlockSpec((1, 1, 3 * CC), lambda t, r: (t, 0, 0)),
            pl.BlockSpec((1, CC, CC), lambda t, r: (t, 0, 0)),
        ],
        out_specs=[
            pl.BlockSpec((1, BR, CC), lambda t, r: (t, r, 0)),
            pl.BlockSpec((1, BR, CC), lambda t, r: (t, r, 0)),
        ],
        out_shape=[
            jax.ShapeDtypeStruct((TT, NN, CC), jnp.float32),
            jax.ShapeDtypeStruct((TT, NN, CC), jnp.float32),
        ],
    )(agg, agg, h, gh1, Wih, bih.reshape(TT, 1, 3 * CC), W1)


def _final(agg1, x1, gh2, h, Wih, bih, ln_g, ln_b):
    def body(a0l_ref, a0h_ref, a1l_ref, a1h_ref, a2l_ref, a2h_ref,
             x1_ref, gh_ref, h_ref, wih_ref,
             bih_ref, lng_ref, lnb_ref, y_ref):
        a_refs = ((a0l_ref, a0h_ref), (a1l_ref, a1h_ref), (a2l_ref, a2h_ref))
        acc = h_ref[...]
        for t in range(TT):
            acc = acc + _gru_combine(a_refs[t][0][0], a_refs[t][1][0],
                                     gh_ref[t], x1_ref[t],
                                     wih_ref[t], bih_ref[t])
        mu = jnp.mean(acc, axis=1, keepdims=True)
        d = acc - mu
        var = jnp.mean(d * d, axis=1, keepdims=True)
        y = d * lax.rsqrt(var + 1e-5) * lng_ref[...] + lnb_ref[...]
        y_ref[...] = jnp.where(y >= 0, y, 0.1 * y)

    def agg_spec(t, half):
        return pl.BlockSpec((1, BR, HC),
                            lambda r, t=t, half=half: (half, t * NB + r, 0))
    return pl.pallas_call(
        body,
        grid=(NB,),
        in_specs=[
            agg_spec(0, 0), agg_spec(0, 1),
            agg_spec(1, 0), agg_spec(1, 1),
            agg_spec(2, 0), agg_spec(2, 1),
            pl.BlockSpec((TT, BR, CC), lambda r: (0, r, 0)),
            pl.BlockSpec((TT, BR, 3 * CC), lambda r: (0, r, 0)),
            pl.BlockSpec((BR, CC), lambda r: (r, 0)),
            pl.BlockSpec((TT, 3 * CC, CC), lambda r: (0, 0, 0)),
            pl.BlockSpec((TT, 3 * CC), lambda r: (0, 0)),
            pl.BlockSpec((1, CC), lambda r: (0, 0)),
            pl.BlockSpec((1, CC), lambda r: (0, 0)),
        ],
        out_specs=pl.BlockSpec((BR, CC), lambda r: (r, 0)),
        out_shape=jax.ShapeDtypeStruct((NN, CC), jnp.float32),
    )(agg1, agg1, agg1, agg1, agg1, agg1, x1, gh2, h, Wih, bih, ln_g, ln_b)


def kernel(h, edge_index, edge_type, W, Wih, Whh, bih, bhh, ln_g, ln_b):
    src = edge_index[0].astype(jnp.int32)
    dst = edge_index[1].astype(jnp.int32)
    typ = edge_type.astype(jnp.int32)
    npad = E_ALLOC - EE
    src_p = jnp.concatenate([src, jnp.zeros((npad,), jnp.int32)])
    dst_p = jnp.concatenate([dst, jnp.full((npad,), TRASH, jnp.int32)])
    typ_p = jnp.concatenate([typ, jnp.zeros((npad,), jnp.int32)])

    m0 = _msg_matmul(h, W[:, 0])
    agg0 = _sc_edge_pass(m0.reshape(2 * TT * NN, HC),
                         src_p, dst_p, typ_p)
    gh1 = _gh_step1(h, Whh, bhh)
    x1, m1 = _gru_and_msg(agg0, h, gh1, Wih, bih, W[:, 1])
    agg1 = _sc_edge_pass(m1.reshape(2 * TT * NN, HC),
                         src_p, dst_p, typ_p)
    gh2 = _gh_step2(x1, Whh, bhh)
    return _final(agg1, x1, gh2, h, Wih, bih,
                  ln_g.reshape(1, CC), ln_b.reshape(1, CC))


# speedup vs baseline: 1.1187x; 1.0440x over previous
import functools

import jax
import jax.numpy as jnp
from jax import lax
from jax.experimental import pallas as pl
from jax.experimental.pallas import tpu as pltpu
from jax.experimental.pallas import tpu_sc as plsc

NN = 10000
EE = 320000
CC = 128
TT = 3
HC = CC // 2

CHUNK = 48
SB = 336
NCH = SB // CHUNK
TILES = 16
EDGES_PER_TILE = 20160
NSB = EDGES_PER_TILE // SB
E_PAD = EDGES_PER_TILE * TILES
E_ALLOC = E_PAD + SB
STRIPE = 1876
SLOTS = STRIPE * TILES
WZ = 28
TRASH = TT * NN

BR = 400
NB = NN // BR
_PREC = lax.Precision.HIGHEST


def _sc_edge_pass(table, src_p, dst_p, typ_p):
    mesh = plsc.VectorSubcoreMesh(core_axis_name="c", subcore_axis_name="s")

    @functools.partial(
        pl.kernel,
        out_type=jax.ShapeDtypeStruct((SLOTS, CC), jnp.float32),
        mesh=mesh,
        scratch_types=[
            pltpu.VMEM((2, SB), jnp.int32),
            pltpu.VMEM((2, SB), jnp.int32),
            pltpu.VMEM((2, SB), jnp.int32),
            pltpu.VMEM((CHUNK,), jnp.int32),
            pltpu.VMEM((CHUNK,), jnp.int32),
            pltpu.VMEM((CHUNK,), jnp.int32),
            pltpu.VMEM((CHUNK,), jnp.int32),
            pltpu.VMEM((2, CHUNK, HC), jnp.float32),
            pltpu.VMEM((WZ, HC), jnp.float32),
            pltpu.VMEM_SHARED((SLOTS, HC), jnp.float32),
            pltpu.SemaphoreType.DMA,
            pltpu.SemaphoreType.DMA,
            pltpu.SemaphoreType.DMA,
            pltpu.SemaphoreType.DMA,
            pltpu.SemaphoreType.DMA,
            pltpu.SemaphoreType.DMA,
        ],
        compiler_params=pltpu.CompilerParams(use_tc_tiling_on_sc=False),
    )
    def k(table_h, src_h, dst_h, typ_h, out_h,
          src_v, dst_v, typ_v, gidx0_v, gidx1_v, didx0_v, didx1_v,
          rows_v, wrows_v,
          acc, sem_i0, sem_i1, sem_g0, sem_g1, sem_s0, sem_s1):
        c = lax.axis_index("c")
        s = lax.axis_index("s")
        gidx = (gidx0_v, gidx1_v)
        didx = (didx0_v, didx1_v)
        sem_i = (sem_i0, sem_i1)
        sem_g = (sem_g0, sem_g1)
        sem_s = (sem_s0, sem_s1)
        stripe_base = s * STRIPE

        def zrow(i, carry):
            for j in range(HC // 16):
                wrows_v[i, pl.ds(j * 16, 16)] = jnp.zeros((16,), jnp.float32)
            return carry
        lax.fori_loop(0, WZ, zrow, 0)

        def zacc(kk, carry):
            pltpu.sync_copy(
                wrows_v, acc.at[pl.ds(stripe_base + kk * WZ, WZ)])
            return carry
        lax.fori_loop(0, STRIPE // WZ, zacc, 0)
        plsc.subcore_barrier()

        def issue_idx_load(sb, buf):
            base = s * EDGES_PER_TILE + sb * SB
            pltpu.async_copy(src_h.at[pl.ds(base, SB)], src_v.at[buf],
                             sem_i[buf])
            pltpu.async_copy(dst_h.at[pl.ds(base, SB)], dst_v.at[buf],
                             sem_i[buf])
            pltpu.async_copy(typ_h.at[pl.ds(base, SB)], typ_v.at[buf],
                             sem_i[buf])

        def wait_idx_load(buf):
            for ref in (src_v, dst_v, typ_v):
                pltpu.make_async_copy(src_h.at[pl.ds(0, SB)], ref.at[buf],
                                      sem_i[buf]).wait()

        def process_superblock(buf, rb_base, state):
            gdesc, sdesc = state
            for ch in range(NCH):
                rb = (rb_base + ch) & 1
                if sdesc[rb] is not None:
                    sdesc[rb].wait()
                    sdesc[rb] = None
                for g in range(CHUNK // 16):
                    off = pl.ds(ch * CHUNK + g * 16, 16)
                    sl = pl.ds(g * 16, 16)
                    t16 = typ_v[buf, off]
                    gidx[rb][sl] = (t16 * NN + src_v[buf, off]) * 2 + c
                    didx[rb][sl] = t16 * NN + dst_v[buf, off]
                if gdesc[rb ^ 1] is not None:
                    gdesc[rb ^ 1].wait()
                    gdesc[rb ^ 1] = None
                    sdesc[rb ^ 1] = pltpu.async_copy(
                        rows_v.at[rb ^ 1], acc.at[didx[rb ^ 1]],
                        sem_s[rb ^ 1], add=True)
                gdesc[rb] = pltpu.async_copy(
                    table_h.at[gidx[rb]], rows_v.at[rb], sem_g[rb])
            return gdesc, sdesc

        def drain(state):
            gdesc, sdesc = state
            for rb in (0, 1):
                if gdesc[rb] is not None:
                    if sdesc[rb] is not None:
                        sdesc[rb].wait()
                    gdesc[rb].wait()
                    sdesc[rb] = pltpu.async_copy(
                        rows_v.at[rb], acc.at[didx[rb]],
                        sem_s[rb], add=True)
                    gdesc[rb] = None
            for rb in (0, 1):
                if sdesc[rb] is not None:
                    sdesc[rb].wait()

        issue_idx_load(0, 0)

        def pair_body(i, carry):
            sb0 = 2 * i
            wait_idx_load(0)
            issue_idx_load(sb0 + 1, 1)
            state = process_superblock(0, 0, ([None, None], [None, None]))
            wait_idx_load(1)
            issue_idx_load(sb0 + 2, 0)
            state = process_superblock(1, NCH & 1, state)
            drain(state)
            return carry
        lax.fori_loop(0, NSB // 2, pair_body, 0)
        wait_idx_load(0)
        plsc.subcore_barrier()

        pltpu.sync_copy(acc.at[pl.ds(stripe_base, STRIPE)],
                        out_h.at[pl.ds(stripe_base, STRIPE),
                                 pl.ds(c * HC, HC)])

    return k(table, src_p, dst_p, typ_p)


def _msg_matmul(h, W0):
    def body(h_ref, w_ref, o_ref):
        o_ref[0] = lax.dot_general(
            h_ref[...], w_ref[0], (((1,), (0,)), ((), ())),
            preferred_element_type=jnp.float32, precision=_PREC)
    return pl.pallas_call(
        body,
        grid=(TT, NB),
        in_specs=[
            pl.BlockSpec((BR, CC), lambda t, r: (r, 0)),
            pl.BlockSpec((1, CC, CC), lambda t, r: (t, 0, 0)),
        ],
        out_specs=pl.BlockSpec((1, BR, CC), lambda t, r: (t, r, 0)),
        out_shape=jax.ShapeDtypeStruct((TT, NN, CC), jnp.float32),
    )(h, W0)


def _gh_step1(h, Whh, bhh):
    def body(h_ref, whh_ref, bhh_ref, o_ref):
        o_ref[0] = lax.dot_general(
            h_ref[...], whh_ref[0], (((1,), (1,)), ((), ())),
            preferred_element_type=jnp.float32,
            precision=_PREC) + bhh_ref[0, 0]
    return pl.pallas_call(
        body,
        grid=(TT, NB),
        in_specs=[
            pl.BlockSpec((BR, CC), lambda t, r: (r, 0)),
            pl.BlockSpec((1, 3 * CC, CC), lambda t, r: (t, 0, 0)),
            pl.BlockSpec((1, 1, 3 * CC), lambda t, r: (t, 0, 0)),
        ],
        out_specs=pl.BlockSpec((1, BR, 3 * CC), lambda t, r: (t, r, 0)),
        out_shape=jax.ShapeDtypeStruct((TT, NN, 3 * CC), jnp.float32),
    )(h, Whh, bhh.reshape(TT, 1, 3 * CC))


def _gh_step2(x1, Whh, bhh):
    def body(x_ref, whh_ref, bhh_ref, o_ref):
        o_ref[0] = lax.dot_general(
            x_ref[0], whh_ref[0], (((1,), (1,)), ((), ())),
            preferred_element_type=jnp.float32,
            precision=_PREC) + bhh_ref[0, 0]
    return pl.pallas_call(
        body,
        grid=(TT, NB),
        in_specs=[
            pl.BlockSpec((1, BR, CC), lambda t, r: (t, r, 0)),
            pl.BlockSpec((1, 3 * CC, CC), lambda t, r: (t, 0, 0)),
            pl.BlockSpec((1, 1, 3 * CC), lambda t, r: (t, 0, 0)),
        ],
        out_specs=pl.BlockSpec((1, BR, 3 * CC), lambda t, r: (t, r, 0)),
        out_shape=jax.ShapeDtypeStruct((TT, NN, 3 * CC), jnp.float32),
    )(x1, Whh, bhh.reshape(TT, 1, 3 * CC))


def _gru_combine(a, gh, hp, wih, bih):
    gi = lax.dot_general(a, wih, (((1,), (1,)), ((), ())),
                         preferred_element_type=jnp.float32,
                         precision=_PREC) + bih
    r = jax.nn.sigmoid(gi[:, :CC] + gh[:, :CC])
    z = jax.nn.sigmoid(gi[:, CC:2 * CC] + gh[:, CC:2 * CC])
    n = jnp.tanh(gi[:, 2 * CC:] + r * gh[:, 2 * CC:])
    return (1.0 - z) * n + z * hp


def _gru_and_msg(agg, h, gh1, Wih, bih, W1):
    def body(a_ref, h_ref, gh_ref, wih_ref, bih_ref, w1_ref,
             x1_ref, m1_ref):
        x1 = _gru_combine(a_ref[...], gh_ref[0], h_ref[...],
                          wih_ref[0], bih_ref[0, 0])
        x1_ref[0] = x1
        m1_ref[0] = lax.dot_general(
            x1, w1_ref[0], (((1,), (0,)), ((), ())),
            preferred_element_type=jnp.float32, precision=_PREC)
    return pl.pallas_call(
        body,
        grid=(TT, NB),
        in_specs=[
            pl.BlockSpec((BR, CC), lambda t, r: (t * NB + r, 0)),
            pl.BlockSpec((BR, CC), lambda t, r: (r, 0)),
            pl.BlockSpec((1, BR, 3 * CC), lambda t, r: (t, r, 0)),
            pl.BlockSpec((1, 3 * CC, CC), lambda t, r: (t, 0, 0)),
            pl.BlockSpec((1, 1, 3 * CC), lambda t, r: (t, 0, 0)),
            pl.BlockSpec((1, CC, CC), lambda t, r: (t, 0, 0)),
        ],
        out_specs=[
            pl.BlockSpec((1, BR, CC), lambda t, r: (t, r, 0)),
            pl.BlockSpec((1, BR, CC), lambda t, r: (t, r, 0)),
        ],
        out_shape=[
            jax.ShapeDtypeStruct((TT, NN, CC), jnp.float32),
            jax.ShapeDtypeStruct((TT, NN, CC), jnp.float32),
        ],
    )(agg, h, gh1, Wih, bih.reshape(TT, 1, 3 * CC), W1)


def _final(agg1, x1, gh2, h, Wih, bih, ln_g, ln_b):
    def body(a0_ref, a1_ref, a2_ref,
             x1_ref, gh_ref, h_ref, wih_ref,
             bih_ref, lng_ref, lnb_ref, y_ref):
        a_refs = (a0_ref, a1_ref, a2_ref)
        acc = h_ref[...]
        for t in range(TT):
            acc = acc + _gru_combine(a_refs[t][...],
                                     gh_ref[t], x1_ref[t],
                                     wih_ref[t], bih_ref[t])
        mu = jnp.mean(acc, axis=1, keepdims=True)
        d = acc - mu
        var = jnp.mean(d * d, axis=1, keepdims=True)
        y = d * lax.rsqrt(var + 1e-5) * lng_ref[...] + lnb_ref[...]
        y_ref[...] = jnp.where(y >= 0, y, 0.1 * y)

    agg_spec = lambda t: pl.BlockSpec((BR, CC), lambda r, t=t: (t * NB + r, 0))
    return pl.pallas_call(
        body,
        grid=(NB,),
        in_specs=[
            agg_spec(0), agg_spec(1), agg_spec(2),
            pl.BlockSpec((TT, BR, CC), lambda r: (0, r, 0)),
            pl.BlockSpec((TT, BR, 3 * CC), lambda r: (0, r, 0)),
            pl.BlockSpec((BR, CC), lambda r: (r, 0)),
            pl.BlockSpec((TT, 3 * CC, CC), lambda r: (0, 0, 0)),
            pl.BlockSpec((TT, 3 * CC), lambda r: (0, 0)),
            pl.BlockSpec((1, CC), lambda r: (0, 0)),
            pl.BlockSpec((1, CC), lambda r: (0, 0)),
        ],
        out_specs=pl.BlockSpec((BR, CC), lambda r: (r, 0)),
        out_shape=jax.ShapeDtypeStruct((NN, CC), jnp.float32),
    )(agg1, agg1, agg1, x1, gh2, h, Wih, bih, ln_g, ln_b)


def kernel(h, edge_index, edge_type, W, Wih, Whh, bih, bhh, ln_g, ln_b):
    src = edge_index[0].astype(jnp.int32)
    dst = edge_index[1].astype(jnp.int32)
    typ = edge_type.astype(jnp.int32)
    npad = E_ALLOC - EE
    src_p = jnp.concatenate([src, jnp.zeros((npad,), jnp.int32)])
    dst_p = jnp.concatenate([dst, jnp.full((npad,), TRASH, jnp.int32)])
    typ_p = jnp.concatenate([typ, jnp.zeros((npad,), jnp.int32)])

    m0 = _msg_matmul(h, W[:, 0])
    agg0 = _sc_edge_pass(m0.reshape(2 * TT * NN, HC),
                         src_p, dst_p, typ_p)
    gh1 = _gh_step1(h, Whh, bhh)
    x1, m1 = _gru_and_msg(agg0, h, gh1, Wih, bih, W[:, 1])
    agg1 = _sc_edge_pass(m1.reshape(2 * TT * NN, HC),
                         src_p, dst_p, typ_p)
    gh2 = _gh_step2(x1, Whh, bhh)
    return _final(agg1, x1, gh2, h, Wih, bih,
                  ln_g.reshape(1, CC), ln_b.reshape(1, CC))
